# baseline (speedup 1.0000x reference)
import functools

import jax
import jax.numpy as jnp
from jax import lax
from jax.experimental import pallas as pl
from jax.experimental.pallas import tpu as pltpu
from jax.experimental.pallas import tpu_sc as plsc

N = 10000
D = 128
C = 64
F = C + D
G = F // 2
E = 320000

NC = 2
NS = 16
K = 64
S = 4
L = 2
CH = 314
EP = NS * CH * K
ROWS_PER_TILE = 632
ACC_ROWS = NS * ROWS_PER_TILE


def _prep_body(logits_ref, h_ref, out_ref):
    j = pl.program_id(0)

    @pl.when(j == 0)
    def _():
        out_ref[:, :C] = jax.nn.softmax(logits_ref[...], axis=-1)
        out_ref[:, C:] = h_ref[:, : G - C]

    @pl.when(j == 1)
    def _():
        out_ref[...] = h_ref[:, G - C:]


def _prep(logits, h):
    grid = 10
    rb = N // grid
    return pl.pallas_call(
        _prep_body,
        grid=(2, grid),
        in_specs=[
            pl.BlockSpec((rb, C), lambda j, i: (i, 0)),
            pl.BlockSpec((rb, D), lambda j, i: (i, 0)),
        ],
        out_specs=pl.BlockSpec((rb, G), lambda j, i: (j * grid + i, 0)),
        out_shape=jax.ShapeDtypeStruct((2 * N, G), jnp.float32),
    )(logits, h)


def _sc_body(tab, src_hbm, dst_hbm, zero_hbm, out_hbm, *scr):
    c = lax.axis_index("c")
    s = lax.axis_index("s")
    src_v, dst_v = scr[0], scr[1]
    bufs = scr[2:2 + S]
    accum = scr[2 + S]
    sgs = scr[3 + S:3 + 2 * S]
    sss = scr[3 + 2 * S:3 + 3 * S]

    def gather(t, m):
        pltpu.async_copy(tab.at[src_v.at[t]], bufs[m], sgs[m])

    def wait_gather(t, m):
        pltpu.make_async_copy(tab.at[src_v.at[t]], bufs[m], sgs[m]).wait()

    def scatter(t, m):
        pltpu.async_copy(bufs[m], accum.at[dst_v.at[t]], sss[m], add=True)

    def wait_scatter(t, m):
        pltpu.make_async_copy(bufs[m], accum.at[dst_v.at[t]], sss[m]).wait()

    pltpu.sync_copy(zero_hbm, accum.at[pl.ds(s * ROWS_PER_TILE, ROWS_PER_TILE)])
    plsc.subcore_barrier()

    pltpu.sync_copy(src_hbm.at[c, s], src_v)
    pltpu.sync_copy(dst_hbm.at[s], dst_v)

    gather(0, 0)
    gather(1, 1)
    gather(2, 2)
    wait_gather(0, 0)
    scatter(0, 0)
    gather(3, 3)
    wait_gather(1, 1)
    scatter(1, 1)
    wait_scatter(0, 0)
    gather(4, 0)
    wait_gather(2, 2)
    scatter(2, 2)
    wait_scatter(1, 1)
    gather(5, 1)
    wait_gather(3, 3)
    scatter(3, 3)

    def group(gi, _):
        t0 = 2 + 4 * gi
        for m in range(4):
            gslot = (2 + m) % 4
            wait_scatter(t0 + m - 4, gslot)
            gather(t0 + m, gslot)
            wait_gather(t0 + m - 2, m)
            scatter(t0 + m - 2, m)
        return 0

    lax.fori_loop(1, (CH - 2) // 4, group, 0)
    wait_gather(CH - 2, (CH - 2) % 4)
    scatter(CH - 2, (CH - 2) % 4)
    wait_gather(CH - 1, (CH - 1) % 4)
    scatter(CH - 1, (CH - 1) % 4)
    for t in range(CH - 4, CH):
        wait_scatter(t, t % 4)

    plsc.subcore_barrier()

    r0 = s * ROWS_PER_TILE
    pltpu.sync_copy(accum.at[pl.ds(r0, ROWS_PER_TILE)],
                    out_hbm.at[c].at[pl.ds(r0, ROWS_PER_TILE)])


def _sc_segment_sum(tab, srcb, dstb, zero):
    mesh = plsc.VectorSubcoreMesh(core_axis_name="c", subcore_axis_name="s")
    f = functools.partial(
        pl.kernel,
        out_type=jax.ShapeDtypeStruct((NC, ACC_ROWS, G), jnp.float32),
        mesh=mesh,
        scratch_types=(
            [pltpu.VMEM((CH, K), jnp.int32),
             pltpu.VMEM((CH, K), jnp.int32)]
            + [pltpu.VMEM((K, G), jnp.float32) for _ in range(S)]
            + [pltpu.VMEM_SHARED((ACC_ROWS, G), jnp.float32)]
            + [pltpu.SemaphoreType.DMA for _ in range(2 * S)]
        ),
        compiler_params=pltpu.CompilerParams(use_tc_tiling_on_sc=False),
    )(_sc_body)
    return f(tab, srcb, dstb, zero)


_GRID = 10
_RB = N // _GRID


def _stats_body(ta_ref, part_ref, norm_ref, f1_ref, f2_ref, ps_ref):
    p = ta_ref[:, :C]
    nrm = norm_ref[...]
    p_bar = part_ref[0, :, :C] * nrm
    eps = 1e-12
    f1 = -jnp.sum(p_bar * jnp.log(p_bar + eps), axis=-1, keepdims=True)
    f2 = jnp.sum(p * (jnp.log(p + eps) - jnp.log(p_bar + eps)),
                 axis=-1, keepdims=True)
    f1_ref[...] = f1
    f2_ref[...] = f2
    m1 = jnp.mean(f1)
    m2 = jnp.mean(f2)
    ps_ref[...] = jnp.stack(
        [m1, jnp.sum((f1 - m1) ** 2), m2, jnp.sum((f2 - m2) ** 2),
         m1, m1, m1, m1]).reshape(1, 1, 8)


def _update_body(ta_ref, tb_ref, part_ref, norm_ref, f1_ref, f2_ref, ps_ref,
                 oldz_ref, tau1_ref, tau2_ref, newh_ref, z_ref):
    ps = ps_ref[:, 0, :]
    mu1 = jnp.mean(ps[:, 0:1])
    mu2 = jnp.mean(ps[:, 2:3])
    var1 = (jnp.sum(ps[:, 1:2]) + _RB * jnp.sum((ps[:, 0:1] - mu1) ** 2)) / N
    var2 = (jnp.sum(ps[:, 3:4]) + _RB * jnp.sum((ps[:, 2:3] - mu2) ** 2)) / N
    nf1 = (f1_ref[...] - mu1) / jnp.sqrt(var1 + 1e-5)
    nf2 = (f2_ref[...] - mu2) / jnp.sqrt(var2 + 1e-5)
    z = (jax.nn.sigmoid(-(nf1 - tau1_ref[0, 0]))
         * jax.nn.sigmoid(-(nf2 - tau2_ref[0, 0])))
    gate = jnp.minimum(oldz_ref[...], z)
    h = jnp.concatenate([ta_ref[:, C:], tb_ref[...]], axis=-1)
    agg = jnp.concatenate([part_ref[0, :, C:], part_ref[1]], axis=-1)
    normagg = jnp.maximum(agg * norm_ref[...], 0.0)
    newh_ref[...] = h + gate * normagg
    z_ref[...] = z


def _finalize(tab, parts, norm, old_z, tau_1, tau_2):
    f1, f2, ps = pl.pallas_call(
        _stats_body,
        grid=(_GRID,),
        in_specs=[
            pl.BlockSpec((_RB, G), lambda i: (i, 0)),
            pl.BlockSpec((2, _RB, G), lambda i: (0, i, 0)),
            pl.BlockSpec((_RB, 1), lambda i: (i, 0)),
        ],
        out_specs=(pl.BlockSpec((_RB, 1), lambda i: (i, 0)),
                   pl.BlockSpec((_RB, 1), lambda i: (i, 0)),
                   pl.BlockSpec((1, 1, 8), lambda i: (i, 0, 0))),
        out_shape=(jax.ShapeDtypeStruct((N, 1), jnp.float32),
                   jax.ShapeDtypeStruct((N, 1), jnp.float32),
                   jax.ShapeDtypeStruct((_GRID, 1, 8), jnp.float32)),
    )(tab, parts, norm)
    new_h, z = pl.pallas_call(
        _update_body,
        grid=(_GRID,),
        in_specs=[
            pl.BlockSpec((_RB, G), lambda i: (i, 0)),
            pl.BlockSpec((_RB, G), lambda i: (_GRID + i, 0)),
            pl.BlockSpec((2, _RB, G), lambda i: (0, i, 0)),
            pl.BlockSpec((_RB, 1), lambda i: (i, 0)),
            pl.BlockSpec((_RB, 1), lambda i: (i, 0)),
            pl.BlockSpec((_RB, 1), lambda i: (i, 0)),
            pl.BlockSpec((_GRID, 1, 8), lambda i: (0, 0, 0)),
            pl.BlockSpec((_RB, 1), lambda i: (i, 0)),
            pl.BlockSpec((1, 1), lambda i: (0, 0)),
            pl.BlockSpec((1, 1), lambda i: (0, 0)),
        ],
        out_specs=(pl.BlockSpec((_RB, D), lambda i: (i, 0)),
                   pl.BlockSpec((_RB, 1), lambda i: (i, 0))),
        out_shape=(jax.ShapeDtypeStruct((N, D), jnp.float32),
                   jax.ShapeDtypeStruct((N, 1), jnp.float32)),
    )(tab, tab, parts, norm, f1, f2, ps, old_z.reshape(N, 1),
      tau_1.reshape(1, 1), tau_2.reshape(1, 1))
    return new_h, z


def kernel(h, logits, old_z, norm, tau_1, tau_2, edge_index):
    tab = _prep(logits, h)
    pad = EP - E
    src = jnp.concatenate([edge_index[0], jnp.zeros((pad,), jnp.int32)])
    dst = jnp.concatenate([edge_index[1], jnp.full((pad,), N, jnp.int32)])
    srcb = jnp.stack([src, src + N]).reshape(NC, NS, CH, K)
    dstb = dst.reshape(NS, CH, K)
    zero = jnp.zeros((ROWS_PER_TILE, G), jnp.float32)
    parts = _sc_segment_sum(tab, srcb, dstb, zero)
    new_h, z = _finalize(tab, parts, norm, old_z, tau_1, tau_2)
    return new_h, z.reshape(N)


# speedup vs baseline: 1.0176x; 1.0176x over previous
import functools

import jax
import jax.numpy as jnp
from jax import lax
from jax.experimental import pallas as pl
from jax.experimental.pallas import tpu as pltpu
from jax.experimental.pallas import tpu_sc as plsc

N = 10000
D = 128
C = 64
F = C + D
G = F // 2
E = 320000

NC = 2
NS = 16
K = 64
S = 4
L = 2
CH = 314
EP = NS * CH * K
ROWS_PER_TILE = 632
ACC_ROWS = NS * ROWS_PER_TILE


def _prep_body(logits_ref, h_ref, out_ref):
    out_ref[0, :, :C] = jax.nn.softmax(logits_ref[...], axis=-1)
    out_ref[0, :, C:] = h_ref[:, : G - C]
    out_ref[1] = h_ref[:, G - C:]


def _prep(logits, h):
    grid = 10
    rb = N // grid
    tabs = pl.pallas_call(
        _prep_body,
        grid=(grid,),
        in_specs=[
            pl.BlockSpec((rb, C), lambda i: (i, 0)),
            pl.BlockSpec((rb, D), lambda i: (i, 0)),
        ],
        out_specs=pl.BlockSpec((2, rb, G), lambda i: (0, i, 0)),
        out_shape=jax.ShapeDtypeStruct((2, N, G), jnp.float32),
    )(logits, h)
    return tabs.reshape(2 * N, G)


def _sc_body(tab, src_hbm, dst_hbm, zero_hbm, out_hbm, *scr):
    c = lax.axis_index("c")
    s = lax.axis_index("s")
    src_v, dst_v = scr[0], scr[1]
    bufs = scr[2:2 + S]
    accum = scr[2 + S]
    sgs = scr[3 + S:3 + 2 * S]
    sss = scr[3 + 2 * S:3 + 3 * S]

    def gather(t, m):
        pltpu.async_copy(tab.at[src_v.at[t]], bufs[m], sgs[m])

    def wait_gather(t, m):
        pltpu.make_async_copy(tab.at[src_v.at[t]], bufs[m], sgs[m]).wait()

    def scatter(t, m):
        pltpu.async_copy(bufs[m], accum.at[dst_v.at[t]], sss[m], add=True)

    def wait_scatter(t, m):
        pltpu.make_async_copy(bufs[m], accum.at[dst_v.at[t]], sss[m]).wait()

    pltpu.sync_copy(zero_hbm, accum.at[pl.ds(s * ROWS_PER_TILE, ROWS_PER_TILE)])
    plsc.subcore_barrier()

    pltpu.sync_copy(src_hbm.at[c, s], src_v)
    pltpu.sync_copy(dst_hbm.at[s], dst_v)

    gather(0, 0)
    gather(1, 1)
    gather(2, 2)
    wait_gather(0, 0)
    scatter(0, 0)
    gather(3, 3)
    wait_gather(1, 1)
    scatter(1, 1)
    wait_scatter(0, 0)
    gather(4, 0)
    wait_gather(2, 2)
    scatter(2, 2)
    wait_scatter(1, 1)
    gather(5, 1)
    wait_gather(3, 3)
    scatter(3, 3)

    def group(gi, _):
        t0 = 2 + 4 * gi
        for m in range(4):
            gslot = (2 + m) % 4
            wait_scatter(t0 + m - 4, gslot)
            gather(t0 + m, gslot)
            wait_gather(t0 + m - 2, m)
            scatter(t0 + m - 2, m)
        return 0

    lax.fori_loop(1, (CH - 2) // 4, group, 0)
    wait_gather(CH - 2, (CH - 2) % 4)
    scatter(CH - 2, (CH - 2) % 4)
    wait_gather(CH - 1, (CH - 1) % 4)
    scatter(CH - 1, (CH - 1) % 4)
    for t in range(CH - 4, CH):
        wait_scatter(t, t % 4)

    plsc.subcore_barrier()

    r0 = s * ROWS_PER_TILE
    pltpu.sync_copy(accum.at[pl.ds(r0, ROWS_PER_TILE)],
                    out_hbm.at[c].at[pl.ds(r0, ROWS_PER_TILE)])


def _sc_segment_sum(tab, srcb, dstb, zero):
    mesh = plsc.VectorSubcoreMesh(core_axis_name="c", subcore_axis_name="s")
    f = functools.partial(
        pl.kernel,
        out_type=jax.ShapeDtypeStruct((NC, ACC_ROWS, G), jnp.float32),
        mesh=mesh,
        scratch_types=(
            [pltpu.VMEM((CH, K), jnp.int32),
             pltpu.VMEM((CH, K), jnp.int32)]
            + [pltpu.VMEM((K, G), jnp.float32) for _ in range(S)]
            + [pltpu.VMEM_SHARED((ACC_ROWS, G), jnp.float32)]
            + [pltpu.SemaphoreType.DMA for _ in range(2 * S)]
        ),
        compiler_params=pltpu.CompilerParams(use_tc_tiling_on_sc=False),
    )(_sc_body)
    return f(tab, srcb, dstb, zero)


_GRID = 10
_RB = N // _GRID


def _stats_body(ta_ref, part_ref, norm_ref, f1_ref, f2_ref, ps_ref):
    p = ta_ref[:, :C]
    nrm = norm_ref[...]
    p_bar = part_ref[0, :, :C] * nrm
    eps = 1e-12
    f1 = -jnp.sum(p_bar * jnp.log(p_bar + eps), axis=-1, keepdims=True)
    f2 = jnp.sum(p * (jnp.log(p + eps) - jnp.log(p_bar + eps)),
                 axis=-1, keepdims=True)
    f1_ref[...] = f1
    f2_ref[...] = f2
    m1 = jnp.mean(f1)
    m2 = jnp.mean(f2)
    ps_ref[...] = jnp.stack(
        [m1, jnp.sum((f1 - m1) ** 2), m2, jnp.sum((f2 - m2) ** 2),
         m1, m1, m1, m1]).reshape(1, 1, 8)


def _update_body(ta_ref, tb_ref, part_ref, norm_ref, f1_ref, f2_ref, ps_ref,
                 oldz_ref, tau1_ref, tau2_ref, newh_ref, z_ref):
    ps = ps_ref[:, 0, :]
    mu1 = jnp.mean(ps[:, 0:1])
    mu2 = jnp.mean(ps[:, 2:3])
    var1 = (jnp.sum(ps[:, 1:2]) + _RB * jnp.sum((ps[:, 0:1] - mu1) ** 2)) / N
    var2 = (jnp.sum(ps[:, 3:4]) + _RB * jnp.sum((ps[:, 2:3] - mu2) ** 2)) / N
    nf1 = (f1_ref[...] - mu1) / jnp.sqrt(var1 + 1e-5)
    nf2 = (f2_ref[...] - mu2) / jnp.sqrt(var2 + 1e-5)
    z = (jax.nn.sigmoid(-(nf1 - tau1_ref[0, 0]))
         * jax.nn.sigmoid(-(nf2 - tau2_ref[0, 0])))
    gate = jnp.minimum(oldz_ref[...], z)
    h = jnp.concatenate([ta_ref[:, C:], tb_ref[...]], axis=-1)
    agg = jnp.concatenate([part_ref[0, :, C:], part_ref[1]], axis=-1)
    normagg = jnp.maximum(agg * norm_ref[...], 0.0)
    newh_ref[...] = h + gate * normagg
    z_ref[...] = z


def _finalize(tab, parts, norm, old_z, tau_1, tau_2):
    f1, f2, ps = pl.pallas_call(
        _stats_body,
        grid=(_GRID,),
        in_specs=[
            pl.BlockSpec((_RB, G), lambda i: (i, 0)),
            pl.BlockSpec((2, _RB, G), lambda i: (0, i, 0)),
            pl.BlockSpec((_RB, 1), lambda i: (i, 0)),
        ],
        out_specs=(pl.BlockSpec((_RB, 1), lambda i: (i, 0)),
                   pl.BlockSpec((_RB, 1), lambda i: (i, 0)),
                   pl.BlockSpec((1, 1, 8), lambda i: (i, 0, 0))),
        out_shape=(jax.ShapeDtypeStruct((N, 1), jnp.float32),
                   jax.ShapeDtypeStruct((N, 1), jnp.float32),
                   jax.ShapeDtypeStruct((_GRID, 1, 8), jnp.float32)),
    )(tab, parts, norm)
    new_h, z = pl.pallas_call(
        _update_body,
        grid=(_GRID,),
        in_specs=[
            pl.BlockSpec((_RB, G), lambda i: (i, 0)),
            pl.BlockSpec((_RB, G), lambda i: (_GRID + i, 0)),
            pl.BlockSpec((2, _RB, G), lambda i: (0, i, 0)),
            pl.BlockSpec((_RB, 1), lambda i: (i, 0)),
            pl.BlockSpec((_RB, 1), lambda i: (i, 0)),
            pl.BlockSpec((_RB, 1), lambda i: (i, 0)),
            pl.BlockSpec((_GRID, 1, 8), lambda i: (0, 0, 0)),
            pl.BlockSpec((_RB, 1), lambda i: (i, 0)),
            pl.BlockSpec((1, 1), lambda i: (0, 0)),
            pl.BlockSpec((1, 1), lambda i: (0, 0)),
        ],
        out_specs=(pl.BlockSpec((_RB, D), lambda i: (i, 0)),
                   pl.BlockSpec((_RB, 1), lambda i: (i, 0))),
        out_shape=(jax.ShapeDtypeStruct((N, D), jnp.float32),
                   jax.ShapeDtypeStruct((N, 1), jnp.float32)),
    )(tab, tab, parts, norm, f1, f2, ps, old_z.reshape(N, 1),
      tau_1.reshape(1, 1), tau_2.reshape(1, 1))
    return new_h, z


def kernel(h, logits, old_z, norm, tau_1, tau_2, edge_index):
    tab = _prep(logits, h)
    pad = EP - E
    src = jnp.concatenate([edge_index[0], jnp.zeros((pad,), jnp.int32)])
    dst = jnp.concatenate([edge_index[1], jnp.full((pad,), N, jnp.int32)])
    srcb = jnp.stack([src, src + N]).reshape(NC, NS, CH, K)
    dstb = dst.reshape(NS, CH, K)
    zero = jnp.zeros((ROWS_PER_TILE, G), jnp.float32)
    parts = _sc_segment_sum(tab, srcb, dstb, zero)
    new_h, z = _finalize(tab, parts, norm, old_z, tau_1, tau_2)
    return new_h, z.reshape(N)


# speedup vs baseline: 1.0315x; 1.0137x over previous
import functools

import jax
import jax.numpy as jnp
from jax import lax
from jax.experimental import pallas as pl
from jax.experimental.pallas import tpu as pltpu
from jax.experimental.pallas import tpu_sc as plsc

N = 10000
D = 128
C = 64
F = C + D
G = F // 2
E = 320000

NC = 2
NS = 16
K = 64
S = 4
L = 2
CH = 314
EP = NS * CH * K
ROWS_PER_TILE = 632
ACC_ROWS = NS * ROWS_PER_TILE


def _prep_body(logits_ref, h_ref, out_ref):
    out_ref[0, :, :C] = jax.nn.softmax(logits_ref[...], axis=-1)
    out_ref[0, :, C:] = h_ref[:, : G - C]
    out_ref[1] = h_ref[:, G - C:]


def _prep(logits, h):
    grid = 10
    rb = N // grid
    tabs = pl.pallas_call(
        _prep_body,
        grid=(grid,),
        in_specs=[
            pl.BlockSpec((rb, C), lambda i: (i, 0)),
            pl.BlockSpec((rb, D), lambda i: (i, 0)),
        ],
        out_specs=pl.BlockSpec((2, rb, G), lambda i: (0, i, 0)),
        out_shape=jax.ShapeDtypeStruct((2, N, G), jnp.float32),
    )(logits, h)
    return tabs.reshape(2 * N, G)


def _sc_body(tab, src_hbm, dst_hbm, zero_hbm, out_hbm, *scr):
    c = lax.axis_index("c")
    s = lax.axis_index("s")
    src_v, dst_v = scr[0], scr[1]
    bufs = scr[2:2 + S]
    accum = scr[2 + S]
    sgs = scr[3 + S:3 + 2 * S]
    sss = scr[3 + 2 * S:3 + 3 * S]

    def gather(t, m):
        pltpu.async_copy(tab.at[src_v.at[t]], bufs[m], sgs[m])

    def wait_gather(t, m):
        pltpu.make_async_copy(tab.at[src_v.at[t]], bufs[m], sgs[m]).wait()

    def scatter(t, m):
        pltpu.async_copy(bufs[m], accum.at[dst_v.at[t]], sss[m], add=True)

    def wait_scatter(t, m):
        pltpu.make_async_copy(bufs[m], accum.at[dst_v.at[t]], sss[m]).wait()

    pltpu.sync_copy(zero_hbm, accum.at[pl.ds(s * ROWS_PER_TILE, ROWS_PER_TILE)])
    plsc.subcore_barrier()

    pltpu.sync_copy(src_hbm.at[c, s], src_v)
    pltpu.sync_copy(dst_hbm.at[s], dst_v)

    gather(0, 0)
    gather(1, 1)
    gather(2, 2)
    wait_gather(0, 0)
    scatter(0, 0)
    gather(3, 3)
    wait_gather(1, 1)
    scatter(1, 1)
    wait_scatter(0, 0)
    gather(4, 0)
    wait_gather(2, 2)
    scatter(2, 2)
    wait_scatter(1, 1)
    gather(5, 1)
    wait_gather(3, 3)
    scatter(3, 3)

    def group(gi, _):
        t0 = 2 + 4 * gi
        for m in range(4):
            gslot = (2 + m) % 4
            wait_scatter(t0 + m - 4, gslot)
            gather(t0 + m, gslot)
            wait_gather(t0 + m - 2, m)
            scatter(t0 + m - 2, m)
        return 0

    lax.fori_loop(1, (CH - 2) // 4, group, 0)
    wait_gather(CH - 2, (CH - 2) % 4)
    scatter(CH - 2, (CH - 2) % 4)
    wait_gather(CH - 1, (CH - 1) % 4)
    scatter(CH - 1, (CH - 1) % 4)
    for t in range(CH - 4, CH):
        wait_scatter(t, t % 4)

    plsc.subcore_barrier()

    r0 = s * ROWS_PER_TILE
    pltpu.sync_copy(accum.at[pl.ds(r0, ROWS_PER_TILE)],
                    out_hbm.at[c].at[pl.ds(r0, ROWS_PER_TILE)])


def _sc_segment_sum(tab, srcb, dstb, zero):
    mesh = plsc.VectorSubcoreMesh(core_axis_name="c", subcore_axis_name="s")
    f = functools.partial(
        pl.kernel,
        out_type=jax.ShapeDtypeStruct((NC, ACC_ROWS, G), jnp.float32),
        mesh=mesh,
        scratch_types=(
            [pltpu.VMEM((CH, K), jnp.int32),
             pltpu.VMEM((CH, K), jnp.int32)]
            + [pltpu.VMEM((K, G), jnp.float32) for _ in range(S)]
            + [pltpu.VMEM_SHARED((ACC_ROWS, G), jnp.float32)]
            + [pltpu.SemaphoreType.DMA for _ in range(2 * S)]
        ),
        compiler_params=pltpu.CompilerParams(use_tc_tiling_on_sc=False),
    )(_sc_body)
    return f(tab, srcb, dstb, zero)


_GRID = 5
_RB = N // _GRID


def _stats_body(ta_ref, part_ref, norm_ref, f1_ref, f2_ref, ps_ref):
    p = ta_ref[:, :C]
    nrm = norm_ref[...]
    p_bar = part_ref[0, :, :C] * nrm
    eps = 1e-12
    f1 = -jnp.sum(p_bar * jnp.log(p_bar + eps), axis=-1, keepdims=True)
    f2 = jnp.sum(p * (jnp.log(p + eps) - jnp.log(p_bar + eps)),
                 axis=-1, keepdims=True)
    f1_ref[...] = f1
    f2_ref[...] = f2
    m1 = jnp.mean(f1)
    m2 = jnp.mean(f2)
    ps_ref[...] = jnp.stack(
        [m1, jnp.sum((f1 - m1) ** 2), m2, jnp.sum((f2 - m2) ** 2),
         m1, m1, m1, m1]).reshape(1, 1, 8)


def _update_body(ta_ref, tb_ref, part_ref, norm_ref, f1_ref, f2_ref, ps_ref,
                 oldz_ref, tau1_ref, tau2_ref, newh_ref, z_ref):
    ps = ps_ref[:, 0, :]
    mu1 = jnp.mean(ps[:, 0:1])
    mu2 = jnp.mean(ps[:, 2:3])
    var1 = (jnp.sum(ps[:, 1:2]) + _RB * jnp.sum((ps[:, 0:1] - mu1) ** 2)) / N
    var2 = (jnp.sum(ps[:, 3:4]) + _RB * jnp.sum((ps[:, 2:3] - mu2) ** 2)) / N
    nf1 = (f1_ref[...] - mu1) / jnp.sqrt(var1 + 1e-5)
    nf2 = (f2_ref[...] - mu2) / jnp.sqrt(var2 + 1e-5)
    z = (jax.nn.sigmoid(-(nf1 - tau1_ref[0, 0]))
         * jax.nn.sigmoid(-(nf2 - tau2_ref[0, 0])))
    gate = jnp.minimum(oldz_ref[...], z)
    h = jnp.concatenate([ta_ref[:, C:], tb_ref[...]], axis=-1)
    agg = jnp.concatenate([part_ref[0, :, C:], part_ref[1]], axis=-1)
    normagg = jnp.maximum(agg * norm_ref[...], 0.0)
    newh_ref[...] = h + gate * normagg
    z_ref[...] = z


def _finalize(tab, parts, norm, old_z, tau_1, tau_2):
    f1, f2, ps = pl.pallas_call(
        _stats_body,
        grid=(_GRID,),
        in_specs=[
            pl.BlockSpec((_RB, G), lambda i: (i, 0)),
            pl.BlockSpec((2, _RB, G), lambda i: (0, i, 0)),
            pl.BlockSpec((_RB, 1), lambda i: (i, 0)),
        ],
        out_specs=(pl.BlockSpec((_RB, 1), lambda i: (i, 0)),
                   pl.BlockSpec((_RB, 1), lambda i: (i, 0)),
                   pl.BlockSpec((1, 1, 8), lambda i: (i, 0, 0))),
        out_shape=(jax.ShapeDtypeStruct((N, 1), jnp.float32),
                   jax.ShapeDtypeStruct((N, 1), jnp.float32),
                   jax.ShapeDtypeStruct((_GRID, 1, 8), jnp.float32)),
    )(tab, parts, norm)
    new_h, z = pl.pallas_call(
        _update_body,
        grid=(_GRID,),
        in_specs=[
            pl.BlockSpec((_RB, G), lambda i: (i, 0)),
            pl.BlockSpec((_RB, G), lambda i: (_GRID + i, 0)),
            pl.BlockSpec((2, _RB, G), lambda i: (0, i, 0)),
            pl.BlockSpec((_RB, 1), lambda i: (i, 0)),
            pl.BlockSpec((_RB, 1), lambda i: (i, 0)),
            pl.BlockSpec((_RB, 1), lambda i: (i, 0)),
            pl.BlockSpec((_GRID, 1, 8), lambda i: (0, 0, 0)),
            pl.BlockSpec((_RB, 1), lambda i: (i, 0)),
            pl.BlockSpec((1, 1), lambda i: (0, 0)),
            pl.BlockSpec((1, 1), lambda i: (0, 0)),
        ],
        out_specs=(pl.BlockSpec((_RB, D), lambda i: (i, 0)),
                   pl.BlockSpec((_RB, 1), lambda i: (i, 0))),
        out_shape=(jax.ShapeDtypeStruct((N, D), jnp.float32),
                   jax.ShapeDtypeStruct((N, 1), jnp.float32)),
    )(tab, tab, parts, norm, f1, f2, ps, old_z.reshape(N, 1),
      tau_1.reshape(1, 1), tau_2.reshape(1, 1))
    return new_h, z


def kernel(h, logits, old_z, norm, tau_1, tau_2, edge_index):
    tab = _prep(logits, h)
    pad = EP - E
    src = jnp.concatenate([edge_index[0], jnp.zeros((pad,), jnp.int32)])
    dst = jnp.concatenate([edge_index[1], jnp.full((pad,), N, jnp.int32)])
    srcb = jnp.stack([src, src + N]).reshape(NC, NS, CH, K)
    dstb = dst.reshape(NS, CH, K)
    zero = jnp.zeros((ROWS_PER_TILE, G), jnp.float32)
    parts = _sc_segment_sum(tab, srcb, dstb, zero)
    new_h, z = _finalize(tab, parts, norm, old_z, tau_1, tau_2)
    return new_h, z.reshape(N)


# speedup vs baseline: 1.0378x; 1.0062x over previous
import functools

import jax
import jax.numpy as jnp
from jax import lax
from jax.experimental import pallas as pl
from jax.experimental.pallas import tpu as pltpu
from jax.experimental.pallas import tpu_sc as plsc

N = 10000
D = 128
C = 64
F = C + D
G = F // 2
E = 320000

NC = 2
NS = 16
K = 64
S = 4
L = 2
CH = 314
EP = NS * CH * K
ROWS_PER_TILE = 632
ACC_ROWS = NS * ROWS_PER_TILE


def _prep_body(logits_ref, h_ref, out_ref):
    out_ref[0, :, :C] = jax.nn.softmax(logits_ref[...], axis=-1)
    out_ref[0, :, C:] = h_ref[:, : G - C]
    out_ref[1] = h_ref[:, G - C:]


def _prep(logits, h):
    grid = 5
    rb = N // grid
    tabs = pl.pallas_call(
        _prep_body,
        grid=(grid,),
        in_specs=[
            pl.BlockSpec((rb, C), lambda i: (i, 0)),
            pl.BlockSpec((rb, D), lambda i: (i, 0)),
        ],
        out_specs=pl.BlockSpec((2, rb, G), lambda i: (0, i, 0)),
        out_shape=jax.ShapeDtypeStruct((2, N, G), jnp.float32),
    )(logits, h)
    return tabs.reshape(2 * N, G)


def _sc_body(tab, src_hbm, dst_hbm, zero_hbm, out_hbm, *scr):
    c = lax.axis_index("c")
    s = lax.axis_index("s")
    src_v, dst_v = scr[0], scr[1]
    bufs = scr[2:2 + S]
    accum = scr[2 + S]
    sgs = scr[3 + S:3 + 2 * S]
    sss = scr[3 + 2 * S:3 + 3 * S]

    def gather(t, m):
        pltpu.async_copy(tab.at[src_v.at[t]], bufs[m], sgs[m])

    def wait_gather(t, m):
        pltpu.make_async_copy(tab.at[src_v.at[t]], bufs[m], sgs[m]).wait()

    def scatter(t, m):
        pltpu.async_copy(bufs[m], accum.at[dst_v.at[t]], sss[m], add=True)

    def wait_scatter(t, m):
        pltpu.make_async_copy(bufs[m], accum.at[dst_v.at[t]], sss[m]).wait()

    pltpu.sync_copy(zero_hbm, accum.at[pl.ds(s * ROWS_PER_TILE, ROWS_PER_TILE)])
    plsc.subcore_barrier()

    pltpu.sync_copy(src_hbm.at[c, s], src_v)
    pltpu.sync_copy(dst_hbm.at[s], dst_v)

    gather(0, 0)
    gather(1, 1)
    gather(2, 2)
    wait_gather(0, 0)
    scatter(0, 0)
    gather(3, 3)
    wait_gather(1, 1)
    scatter(1, 1)
    wait_scatter(0, 0)
    gather(4, 0)
    wait_gather(2, 2)
    scatter(2, 2)
    wait_scatter(1, 1)
    gather(5, 1)
    wait_gather(3, 3)
    scatter(3, 3)

    def group(gi, _):
        t0 = 2 + 4 * gi
        for m in range(4):
            gslot = (2 + m) % 4
            wait_scatter(t0 + m - 4, gslot)
            gather(t0 + m, gslot)
            wait_gather(t0 + m - 2, m)
            scatter(t0 + m - 2, m)
        return 0

    lax.fori_loop(1, (CH - 2) // 4, group, 0)
    wait_gather(CH - 2, (CH - 2) % 4)
    scatter(CH - 2, (CH - 2) % 4)
    wait_gather(CH - 1, (CH - 1) % 4)
    scatter(CH - 1, (CH - 1) % 4)
    for t in range(CH - 4, CH):
        wait_scatter(t, t % 4)

    plsc.subcore_barrier()

    r0 = s * ROWS_PER_TILE
    pltpu.sync_copy(accum.at[pl.ds(r0, ROWS_PER_TILE)],
                    out_hbm.at[c].at[pl.ds(r0, ROWS_PER_TILE)])


def _sc_segment_sum(tab, srcb, dstb, zero):
    mesh = plsc.VectorSubcoreMesh(core_axis_name="c", subcore_axis_name="s")
    f = functools.partial(
        pl.kernel,
        out_type=jax.ShapeDtypeStruct((NC, ACC_ROWS, G), jnp.float32),
        mesh=mesh,
        scratch_types=(
            [pltpu.VMEM((CH, K), jnp.int32),
             pltpu.VMEM((CH, K), jnp.int32)]
            + [pltpu.VMEM((K, G), jnp.float32) for _ in range(S)]
            + [pltpu.VMEM_SHARED((ACC_ROWS, G), jnp.float32)]
            + [pltpu.SemaphoreType.DMA for _ in range(2 * S)]
        ),
        compiler_params=pltpu.CompilerParams(use_tc_tiling_on_sc=False),
    )(_sc_body)
    return f(tab, srcb, dstb, zero)


_GRID = 5
_RB = N // _GRID


def _stats_body(ta_ref, part_ref, norm_ref, f1_ref, f2_ref, ps_ref):
    p = ta_ref[:, :C]
    nrm = norm_ref[...]
    p_bar = part_ref[0, :, :C] * nrm
    eps = 1e-12
    f1 = -jnp.sum(p_bar * jnp.log(p_bar + eps), axis=-1, keepdims=True)
    f2 = jnp.sum(p * (jnp.log(p + eps) - jnp.log(p_bar + eps)),
                 axis=-1, keepdims=True)
    f1_ref[...] = f1
    f2_ref[...] = f2
    m1 = jnp.mean(f1)
    m2 = jnp.mean(f2)
    ps_ref[...] = jnp.stack(
        [m1, jnp.sum((f1 - m1) ** 2), m2, jnp.sum((f2 - m2) ** 2),
         m1, m1, m1, m1]).reshape(1, 1, 8)


def _update_body(ta_ref, tb_ref, part_ref, norm_ref, f1_ref, f2_ref, ps_ref,
                 oldz_ref, tau1_ref, tau2_ref, newh_ref, z_ref):
    ps = ps_ref[:, 0, :]
    mu1 = jnp.mean(ps[:, 0:1])
    mu2 = jnp.mean(ps[:, 2:3])
    var1 = (jnp.sum(ps[:, 1:2]) + _RB * jnp.sum((ps[:, 0:1] - mu1) ** 2)) / N
    var2 = (jnp.sum(ps[:, 3:4]) + _RB * jnp.sum((ps[:, 2:3] - mu2) ** 2)) / N
    nf1 = (f1_ref[...] - mu1) / jnp.sqrt(var1 + 1e-5)
    nf2 = (f2_ref[...] - mu2) / jnp.sqrt(var2 + 1e-5)
    z = (jax.nn.sigmoid(-(nf1 - tau1_ref[0, 0]))
         * jax.nn.sigmoid(-(nf2 - tau2_ref[0, 0])))
    gate = jnp.minimum(oldz_ref[...], z)
    h = jnp.concatenate([ta_ref[:, C:], tb_ref[...]], axis=-1)
    agg = jnp.concatenate([part_ref[0, :, C:], part_ref[1]], axis=-1)
    normagg = jnp.maximum(agg * norm_ref[...], 0.0)
    newh_ref[...] = h + gate * normagg
    z_ref[...] = z


def _finalize(tab, parts, norm, old_z, tau_1, tau_2):
    f1, f2, ps = pl.pallas_call(
        _stats_body,
        grid=(_GRID,),
        in_specs=[
            pl.BlockSpec((_RB, G), lambda i: (i, 0)),
            pl.BlockSpec((2, _RB, G), lambda i: (0, i, 0)),
            pl.BlockSpec((_RB, 1), lambda i: (i, 0)),
        ],
        out_specs=(pl.BlockSpec((_RB, 1), lambda i: (i, 0)),
                   pl.BlockSpec((_RB, 1), lambda i: (i, 0)),
                   pl.BlockSpec((1, 1, 8), lambda i: (i, 0, 0))),
        out_shape=(jax.ShapeDtypeStruct((N, 1), jnp.float32),
                   jax.ShapeDtypeStruct((N, 1), jnp.float32),
                   jax.ShapeDtypeStruct((_GRID, 1, 8), jnp.float32)),
    )(tab, parts, norm)
    new_h, z = pl.pallas_call(
        _update_body,
        grid=(_GRID,),
        in_specs=[
            pl.BlockSpec((_RB, G), lambda i: (i, 0)),
            pl.BlockSpec((_RB, G), lambda i: (_GRID + i, 0)),
            pl.BlockSpec((2, _RB, G), lambda i: (0, i, 0)),
            pl.BlockSpec((_RB, 1), lambda i: (i, 0)),
            pl.BlockSpec((_RB, 1), lambda i: (i, 0)),
            pl.BlockSpec((_RB, 1), lambda i: (i, 0)),
            pl.BlockSpec((_GRID, 1, 8), lambda i: (0, 0, 0)),
            pl.BlockSpec((_RB, 1), lambda i: (i, 0)),
            pl.BlockSpec((1, 1), lambda i: (0, 0)),
            pl.BlockSpec((1, 1), lambda i: (0, 0)),
        ],
        out_specs=(pl.BlockSpec((_RB, D), lambda i: (i, 0)),
                   pl.BlockSpec((_RB, 1), lambda i: (i, 0))),
        out_shape=(jax.ShapeDtypeStruct((N, D), jnp.float32),
                   jax.ShapeDtypeStruct((N, 1), jnp.float32)),
    )(tab, tab, parts, norm, f1, f2, ps, old_z.reshape(N, 1),
      tau_1.reshape(1, 1), tau_2.reshape(1, 1))
    return new_h, z


def kernel(h, logits, old_z, norm, tau_1, tau_2, edge_index):
    tab = _prep(logits, h)
    pad = EP - E
    src = jnp.concatenate([edge_index[0], jnp.zeros((pad,), jnp.int32)])
    dst = jnp.concatenate([edge_index[1], jnp.full((pad,), N, jnp.int32)])
    srcb = jnp.stack([src, src + N]).reshape(NC, NS, CH, K)
    dstb = dst.reshape(NS, CH, K)
    zero = jnp.zeros((ROWS_PER_TILE, G), jnp.float32)
    parts = _sc_segment_sum(tab, srcb, dstb, zero)
    new_h, z = _finalize(tab, parts, norm, old_z, tau_1, tau_2)
    return new_h, z.reshape(N)
